# baseline (speedup 1.0000x reference)
import functools

import jax
import jax.numpy as jnp
from jax import lax
from jax.experimental import pallas as pl
from jax.experimental.pallas import tpu as pltpu
from jax.experimental.pallas import tpu_sc as plsc

H = 512
W = 512
C = 3
PH = 64
PW = 64
NB = 64
NS = 2
NK = NB * NS
PADH = 576
PADW = 576
WIN = PW + 8

NUM_WORKERS = 32

_vector_mesh = plsc.VectorSubcoreMesh(core_axis_name="c", subcore_axis_name="s")
_sc_params = pltpu.CompilerParams(
    use_tc_tiling_on_sc=False, needs_layout_passes=False)


@functools.partial(
    pl.kernel,
    out_type=jax.ShapeDtypeStruct((C, PADH, PADW), jnp.float32),
    mesh=_vector_mesh,
    scratch_types=[
        pltpu.VMEM((C, 16, PADW), jnp.float32),
        pltpu.VMEM((C, 2, PADW), jnp.float32),
        pltpu.SemaphoreType.DMA,
        pltpu.SemaphoreType.DMA,
    ],
    compiler_params=_sc_params,
)
def _pad_kernel(img_hbm, pad_hbm, rbuf, wbuf, sem_in, sem_out):
    wid = lax.axis_index("s") * 2 + lax.axis_index("c")

    r0 = wid * 16
    r2 = wid * 2
    in_copies = []
    for c in range(C):
        in_copies.append(pltpu.async_copy(
            img_hbm.at[c, pl.ds(r0, 16), :],
            rbuf.at[c, :, pl.ds(0, W)], sem_in))
        in_copies.append(pltpu.async_copy(
            img_hbm.at[c, pl.ds(r0, 16), pl.ds(0, PW)],
            rbuf.at[c, :, pl.ds(W, PW)], sem_in))
        in_copies.append(pltpu.async_copy(
            img_hbm.at[c, pl.ds(r2, 2), :],
            wbuf.at[c, :, pl.ds(0, W)], sem_in))
        in_copies.append(pltpu.async_copy(
            img_hbm.at[c, pl.ds(r2, 2), pl.ds(0, PW)],
            wbuf.at[c, :, pl.ds(W, PW)], sem_in))
    for cpy in in_copies:
        cpy.wait()
    out_copies = []
    for c in range(C):
        out_copies.append(pltpu.async_copy(
            rbuf.at[c], pad_hbm.at[c, pl.ds(r0, 16), :], sem_out))
        out_copies.append(pltpu.async_copy(
            wbuf.at[c], pad_hbm.at[c, pl.ds(H + r2, 2), :], sem_out))
    for cpy in out_copies:
        cpy.wait()


@functools.partial(
    pl.kernel,
    out_type=jax.ShapeDtypeStruct((NB, NS * C, PH, PW), jnp.float32),
    mesh=_vector_mesh,
    scratch_types=[
        pltpu.VMEM((NK,), jnp.int32),
        pltpu.VMEM((NK,), jnp.int32),
        pltpu.VMEM((PH, WIN), jnp.float32),
        pltpu.VMEM((PH, WIN), jnp.float32),
        pltpu.VMEM((PH, WIN), jnp.float32),
        pltpu.VMEM((PH, WIN), jnp.float32),
        pltpu.VMEM((PH, PW), jnp.float32),
        pltpu.VMEM((PH, PW), jnp.float32),
        pltpu.VMEM((PH, PW), jnp.float32),
        pltpu.VMEM((PH, PW), jnp.float32),
        pltpu.SemaphoreType.DMA,
        pltpu.SemaphoreType.DMA,
    ],
    compiler_params=_sc_params,
)
def _gather_kernel(pad_hbm, ys_hbm, xs_hbm, out_hbm, ys_v, xs_v,
                   win0, win1, win2, win3, opl0, opl1, opl2, opl3,
                   sem_in, sem_out):
    wid = lax.axis_index("s") * 2 + lax.axis_index("c")

    cy = pltpu.async_copy(ys_hbm, ys_v, sem_in)
    cx = pltpu.async_copy(xs_hbm, xs_v, sem_in)
    cy.wait()
    cx.wait()

    grp = 16 * (wid // 4)
    ys_grp = ys_v[pl.ds(grp, 16)]
    xs_grp = xs_v[pl.ds(grp, 16)]
    lane = lax.iota(jnp.int32, 16)
    lane_base = (wid % 4) * 4

    def coords(u, s):
        t = u * NS + s
        y = jnp.sum(jnp.where(lane == lane_base + t, ys_grp, 0))
        x = jnp.sum(jnp.where(lane == lane_base + t, xs_grp, 0))
        xa = pl.multiple_of(lax.bitwise_and(x, -8), 8)
        return y, xa, x - lax.bitwise_and(x, -8)

    wins = (win0, win1, win2, win3)
    opls = (opl0, opl1, opl2, opl3)
    NBUF = 4
    NT = NS * C * 2

    col_const = [16 * v + lane for v in range(PW // 16)]

    tasks = []
    for u in range(2):
        b = wid * 2 + u
        for s in range(NS):
            y, xa, d = coords(u, s)
            d_vec = jnp.full((16,), d, dtype=jnp.int32)
            src_cols = [d_vec + cc for cc in col_const]
            for c in range(C):
                tasks.append((b, s * C + c, y, xa, src_cols))

    def fire_in(t):
        b, p, y, xa, _ = tasks[t]
        return pltpu.async_copy(
            pad_hbm.at[p % C, pl.ds(y, PH), pl.ds(xa, WIN)],
            wins[t % NBUF], sem_in)

    in_copies = [fire_in(t) for t in range(NBUF)]
    out_copies = []
    for t in range(NT):
        r = t % NBUF
        in_copies[t].wait()
        if t >= NBUF:
            out_copies[t - NBUF].wait()
        src_cols = tasks[t][4]

        def body(i, i_vec, r=r, src_cols=src_cols):
            for v in range(PW // 16):
                x = plsc.load_gather(wins[r], [i_vec, src_cols[v]])
                opls[r][i, pl.ds(16 * v, 16)] = x
            return i_vec + 1

        plsc.parallel_loop(
            0, PH, step=1, unroll=8,
            carry=jnp.zeros((16,), dtype=jnp.int32))(body)
        b, p = tasks[t][0], tasks[t][1]
        out_copies.append(
            pltpu.async_copy(opls[r], out_hbm.at[b, p], sem_out))
        if t + NBUF < NT:
            in_copies.append(fire_in(t + NBUF))
    for cpy in out_copies[NT - NBUF:]:
        cpy.wait()


def kernel(img, dummy, ys, xs):
    del dummy
    imgp = jnp.transpose(img.reshape(H, W, C), (2, 0, 1))
    pad = _pad_kernel(imgp)
    outp = _gather_kernel(pad, ys, xs)
    return jnp.transpose(outp, (0, 2, 3, 1))


# speedup vs baseline: 1.0265x; 1.0265x over previous
import functools

import jax
import jax.numpy as jnp
from jax import lax
from jax.experimental import pallas as pl
from jax.experimental.pallas import tpu as pltpu
from jax.experimental.pallas import tpu_sc as plsc

H = 512
W = 512
C = 3
PH = 64
PW = 64
NB = 64
NS = 2
NK = NB * NS
PADH = 576
PADW = 576
WIN = PW + 8

NUM_WORKERS = 32

_vector_mesh = plsc.VectorSubcoreMesh(core_axis_name="c", subcore_axis_name="s")
_sc_params = pltpu.CompilerParams(
    use_tc_tiling_on_sc=False, needs_layout_passes=False)


@functools.partial(
    pl.kernel,
    out_type=jax.ShapeDtypeStruct((C, PADH, PADW), jnp.float32),
    mesh=_vector_mesh,
    scratch_types=[
        pltpu.VMEM((C, 16, PADW), jnp.float32),
        pltpu.VMEM((C, 2, PADW), jnp.float32),
        pltpu.SemaphoreType.DMA,
        pltpu.SemaphoreType.DMA,
    ],
    compiler_params=_sc_params,
)
def _pad_kernel(img_hbm, pad_hbm, rbuf, wbuf, sem_in, sem_out):
    wid = lax.axis_index("s") * 2 + lax.axis_index("c")

    r0 = wid * 16
    r2 = wid * 2
    in_copies = []
    for c in range(C):
        in_copies.append(pltpu.async_copy(
            img_hbm.at[c, pl.ds(r0, 16), :],
            rbuf.at[c, :, pl.ds(0, W)], sem_in))
        in_copies.append(pltpu.async_copy(
            img_hbm.at[c, pl.ds(r0, 16), pl.ds(0, PW)],
            rbuf.at[c, :, pl.ds(W, PW)], sem_in))
        in_copies.append(pltpu.async_copy(
            img_hbm.at[c, pl.ds(r2, 2), :],
            wbuf.at[c, :, pl.ds(0, W)], sem_in))
        in_copies.append(pltpu.async_copy(
            img_hbm.at[c, pl.ds(r2, 2), pl.ds(0, PW)],
            wbuf.at[c, :, pl.ds(W, PW)], sem_in))
    for cpy in in_copies:
        cpy.wait()
    out_copies = []
    for c in range(C):
        out_copies.append(pltpu.async_copy(
            rbuf.at[c], pad_hbm.at[c, pl.ds(r0, 16), :], sem_out))
        out_copies.append(pltpu.async_copy(
            wbuf.at[c], pad_hbm.at[c, pl.ds(H + r2, 2), :], sem_out))
    for cpy in out_copies:
        cpy.wait()


@functools.partial(
    pl.kernel,
    out_type=jax.ShapeDtypeStruct((NB, NS * C, PH, PW), jnp.float32),
    mesh=_vector_mesh,
    scratch_types=[
        pltpu.VMEM((NK,), jnp.int32),
        pltpu.VMEM((NK,), jnp.int32),
        pltpu.VMEM((NS * C * 2, PH, WIN), jnp.float32),
        pltpu.VMEM((PH, PW), jnp.float32),
        pltpu.VMEM((PH, PW), jnp.float32),
        pltpu.VMEM((PH, PW), jnp.float32),
        pltpu.VMEM((PH, PW), jnp.float32),
        pltpu.SemaphoreType.DMA,
        pltpu.SemaphoreType.DMA,
    ],
    compiler_params=_sc_params,
)
def _gather_kernel(pad_hbm, ys_hbm, xs_hbm, out_hbm, ys_v, xs_v,
                   winb, opl0, opl1, opl2, opl3,
                   sem_in, sem_out):
    wid = lax.axis_index("s") * 2 + lax.axis_index("c")

    cy = pltpu.async_copy(ys_hbm, ys_v, sem_in)
    cx = pltpu.async_copy(xs_hbm, xs_v, sem_in)
    cy.wait()
    cx.wait()

    grp = 16 * (wid // 4)
    ys_grp = ys_v[pl.ds(grp, 16)]
    xs_grp = xs_v[pl.ds(grp, 16)]
    lane = lax.iota(jnp.int32, 16)
    lane_base = (wid % 4) * 4

    def coords(u, s):
        t = u * NS + s
        y = jnp.sum(jnp.where(lane == lane_base + t, ys_grp, 0))
        x = jnp.sum(jnp.where(lane == lane_base + t, xs_grp, 0))
        xa = pl.multiple_of(lax.bitwise_and(x, -8), 8)
        return y, xa, x - lax.bitwise_and(x, -8)

    opls = (opl0, opl1, opl2, opl3)
    NBUF = 4
    NT = NS * C * 2

    col_const = [16 * v + lane for v in range(PW // 16)]

    tasks = []
    for u in range(2):
        b = wid * 2 + u
        for s in range(NS):
            y, xa, d = coords(u, s)
            d_vec = jnp.full((16,), d, dtype=jnp.int32)
            src_cols = [d_vec + cc for cc in col_const]
            for c in range(C):
                tasks.append((b, s * C + c, y, xa, src_cols))

    def fire_in(t):
        b, p, y, xa, _ = tasks[t]
        return pltpu.async_copy(
            pad_hbm.at[p % C, pl.ds(y, PH), pl.ds(xa, WIN)],
            winb.at[t], sem_in)

    in_copies = [fire_in(t) for t in range(NT)]
    out_copies = []
    for t in range(NT):
        r = t % NBUF
        in_copies[t].wait()
        if t >= NBUF:
            out_copies[t - NBUF].wait()
        src_cols = tasks[t][4]

        def body(i, i_vec, t=t, r=r, src_cols=src_cols):
            for v in range(PW // 16):
                x = plsc.load_gather(winb.at[t], [i_vec, src_cols[v]])
                opls[r][i, pl.ds(16 * v, 16)] = x
            return i_vec + 1

        plsc.parallel_loop(
            0, PH, step=1, unroll=4,
            carry=jnp.zeros((16,), dtype=jnp.int32))(body)
        b, p = tasks[t][0], tasks[t][1]
        out_copies.append(
            pltpu.async_copy(opls[r], out_hbm.at[b, p], sem_out))
    for cpy in out_copies[NT - NBUF:]:
        cpy.wait()


def kernel(img, dummy, ys, xs):
    del dummy
    imgp = jnp.transpose(img.reshape(H, W, C), (2, 0, 1))
    pad = _pad_kernel(imgp)
    outp = _gather_kernel(pad, ys, xs)
    return jnp.transpose(outp, (0, 2, 3, 1))


# speedup vs baseline: 1.0430x; 1.0161x over previous
import functools

import jax
import jax.numpy as jnp
from jax import lax
from jax.experimental import pallas as pl
from jax.experimental.pallas import tpu as pltpu
from jax.experimental.pallas import tpu_sc as plsc

H = 512
W = 512
C = 3
PH = 64
PW = 64
NB = 64
NS = 2
NK = NB * NS
PADH = 576
PADW = 576
WIN = PW + 8

NUM_WORKERS = 32

_vector_mesh = plsc.VectorSubcoreMesh(core_axis_name="c", subcore_axis_name="s")
_sc_params = pltpu.CompilerParams(
    use_tc_tiling_on_sc=False, needs_layout_passes=False)


@functools.partial(
    pl.kernel,
    out_type=jax.ShapeDtypeStruct((C, PADH, PADW), jnp.float32),
    mesh=_vector_mesh,
    scratch_types=[
        pltpu.VMEM((C, 16, PADW), jnp.float32),
        pltpu.VMEM((C, 2, PADW), jnp.float32),
        pltpu.SemaphoreType.DMA,
        pltpu.SemaphoreType.DMA,
    ],
    compiler_params=_sc_params,
)
def _pad_kernel(img_hbm, pad_hbm, rbuf, wbuf, sem_in, sem_out):
    wid = lax.axis_index("s") * 2 + lax.axis_index("c")

    r0 = wid * 16
    r2 = wid * 2
    in_copies = []
    for c in range(C):
        in_copies.append((
            pltpu.async_copy(
                img_hbm.at[c, pl.ds(r0, 16), :],
                rbuf.at[c, :, pl.ds(0, W)], sem_in),
            pltpu.async_copy(
                img_hbm.at[c, pl.ds(r0, 16), pl.ds(0, PW)],
                rbuf.at[c, :, pl.ds(W, PW)], sem_in),
            pltpu.async_copy(
                img_hbm.at[c, pl.ds(r2, 2), :],
                wbuf.at[c, :, pl.ds(0, W)], sem_in),
            pltpu.async_copy(
                img_hbm.at[c, pl.ds(r2, 2), pl.ds(0, PW)],
                wbuf.at[c, :, pl.ds(W, PW)], sem_in),
        ))
    out_copies = []
    for c in range(C):
        rb, rw, wb, ww = in_copies[c]
        rb.wait()
        rw.wait()
        out_copies.append(pltpu.async_copy(
            rbuf.at[c], pad_hbm.at[c, pl.ds(r0, 16), :], sem_out))
        wb.wait()
        ww.wait()
        out_copies.append(pltpu.async_copy(
            wbuf.at[c], pad_hbm.at[c, pl.ds(H + r2, 2), :], sem_out))
    for cpy in out_copies:
        cpy.wait()


@functools.partial(
    pl.kernel,
    out_type=jax.ShapeDtypeStruct((NB, NS * C, PH, PW), jnp.float32),
    mesh=_vector_mesh,
    scratch_types=[
        pltpu.VMEM((NK,), jnp.int32),
        pltpu.VMEM((NK,), jnp.int32),
        pltpu.VMEM((NS * C * 2, PH, WIN), jnp.float32),
        pltpu.VMEM((PH, PW), jnp.float32),
        pltpu.VMEM((PH, PW), jnp.float32),
        pltpu.VMEM((PH, PW), jnp.float32),
        pltpu.VMEM((PH, PW), jnp.float32),
        pltpu.SemaphoreType.DMA,
        pltpu.SemaphoreType.DMA,
    ],
    compiler_params=_sc_params,
)
def _gather_kernel(pad_hbm, ys_hbm, xs_hbm, out_hbm, ys_v, xs_v,
                   winb, opl0, opl1, opl2, opl3,
                   sem_in, sem_out):
    wid = lax.axis_index("s") * 2 + lax.axis_index("c")

    cy = pltpu.async_copy(ys_hbm, ys_v, sem_in)
    cx = pltpu.async_copy(xs_hbm, xs_v, sem_in)
    cy.wait()
    cx.wait()

    grp = 16 * (wid // 4)
    ys_grp = ys_v[pl.ds(grp, 16)]
    xs_grp = xs_v[pl.ds(grp, 16)]
    lane = lax.iota(jnp.int32, 16)
    lane_base = (wid % 4) * 4

    def coords(u, s):
        t = u * NS + s
        y = jnp.sum(jnp.where(lane == lane_base + t, ys_grp, 0))
        x = jnp.sum(jnp.where(lane == lane_base + t, xs_grp, 0))
        xa = pl.multiple_of(lax.bitwise_and(x, -8), 8)
        return y, xa, x - lax.bitwise_and(x, -8)

    opls = (opl0, opl1, opl2, opl3)
    NBUF = 4
    NT = NS * C * 2

    col_const = [16 * v + lane for v in range(PW // 16)]

    tasks = []
    for u in range(2):
        b = wid * 2 + u
        for s in range(NS):
            y, xa, d = coords(u, s)
            d_vec = jnp.full((16,), d, dtype=jnp.int32)
            src_cols = [d_vec + cc for cc in col_const]
            for c in range(C):
                tasks.append((b, s * C + c, y, xa, src_cols))

    def fire_in(t):
        b, p, y, xa, _ = tasks[t]
        return pltpu.async_copy(
            pad_hbm.at[p % C, pl.ds(y, PH), pl.ds(xa, WIN)],
            winb.at[t], sem_in)

    in_copies = [fire_in(t) for t in range(NT)]
    out_copies = []
    for t in range(NT):
        r = t % NBUF
        in_copies[t].wait()
        if t >= NBUF:
            out_copies[t - NBUF].wait()
        src_cols = tasks[t][4]

        def body(i, i_vec, t=t, r=r, src_cols=src_cols):
            for v in range(PW // 16):
                x = plsc.load_gather(winb.at[t], [i_vec, src_cols[v]])
                opls[r][i, pl.ds(16 * v, 16)] = x
            return i_vec + 1

        plsc.parallel_loop(
            0, PH, step=1, unroll=4,
            carry=jnp.zeros((16,), dtype=jnp.int32))(body)
        b, p = tasks[t][0], tasks[t][1]
        out_copies.append(
            pltpu.async_copy(opls[r], out_hbm.at[b, p], sem_out))
    for cpy in out_copies[NT - NBUF:]:
        cpy.wait()


def kernel(img, dummy, ys, xs):
    del dummy
    imgp = jnp.transpose(img.reshape(H, W, C), (2, 0, 1))
    pad = _pad_kernel(imgp)
    outp = _gather_kernel(pad, ys, xs)
    return jnp.transpose(outp, (0, 2, 3, 1))
